# baseline (speedup 1.0000x reference)
import functools

import jax
import jax.numpy as jnp
from jax import lax
from jax.experimental import pallas as pl
from jax.experimental.pallas import tpu as pltpu
from jax.experimental.pallas import tpu_sc as plsc

N = 10000
E = 320000
B = 128
DV = 128
DE = 16
DU = 64

NC = 2
NS = 16
NW = NC * NS
EW = E // NW
CL = 125
NCHUNK = EW // CL
KC = 16
NG = NCHUNK // KC
GE = KC * CL
NWR = 10
NR = N // NWR
KC2 = 8
GE2 = KC2 * CL
NG2 = NCHUNK // KC2

BN = 2000
BE = 8000

_f32 = jnp.float32


def _prep_body(x_ref, batch_ref, u_ref, w1a_ref, w1b_ref, w1u_ref,
               xa_ref, xb_ref):
    ug = jnp.dot(u_ref[...], w1u_ref[...], preferred_element_type=_f32)
    gids = lax.broadcasted_iota(jnp.int32, (1, B), 1)
    oh = (batch_ref[...] == gids).astype(_f32)
    xa_ref[...] = (jnp.dot(x_ref[...], w1a_ref[...], preferred_element_type=_f32)
                   + jnp.dot(oh, ug, preferred_element_type=_f32))
    xb_ref[...] = jnp.dot(x_ref[...], w1b_ref[...], preferred_element_type=_f32)


def _tc_prep(x, batch2d, u, w1a, w1b, w1u):
    return pl.pallas_call(
        _prep_body,
        grid=(N // BN,),
        in_specs=[
            pl.BlockSpec((BN, DV), lambda i: (i, 0)),
            pl.BlockSpec((BN, 1), lambda i: (i, 0)),
            pl.BlockSpec((B, DU), lambda i: (0, 0)),
            pl.BlockSpec((DV, DE), lambda i: (0, 0)),
            pl.BlockSpec((DV, DE), lambda i: (0, 0)),
            pl.BlockSpec((DU, DE), lambda i: (0, 0)),
        ],
        out_specs=[
            pl.BlockSpec((BN, DE), lambda i: (i, 0)),
            pl.BlockSpec((BN, DE), lambda i: (i, 0)),
        ],
        out_shape=[
            jax.ShapeDtypeStruct((N, DE), _f32),
            jax.ShapeDtypeStruct((N, DE), _f32),
        ],
    )(x, batch2d, u, w1a, w1b, w1u)


def _sc_gather(xa, xb, rowm, colm, eat):
    mesh = plsc.VectorSubcoreMesh(core_axis_name="c", subcore_axis_name="s",
                                  num_cores=NC, num_subcores=NS)

    @functools.partial(
        pl.kernel,
        out_type=(jax.ShapeDtypeStruct((E, DE), _f32),
                  jax.ShapeDtypeStruct((E, DE), _f32),
                  jax.ShapeDtypeStruct((E, DE), _f32)),
        mesh=mesh,
        scratch_types=[
            pltpu.VMEM((KC2, CL), jnp.int32),
            pltpu.VMEM((KC2, CL), jnp.int32),
            pltpu.VMEM((GE2, DE), _f32),
            pltpu.VMEM((GE2, DE), _f32),
            pltpu.VMEM((DE, GE2), _f32),
            pltpu.VMEM((GE2, DE), _f32),
            pltpu.SemaphoreType.DMA,
            pltpu.SemaphoreType.DMA,
            pltpu.SemaphoreType.DMA,
        ],
        compiler_params=pltpu.CompilerParams(use_tc_tiling_on_sc=False, needs_layout_passes=False),
    )
    def k(xa_hbm, xb_hbm, rowm_hbm, colm_hbm, eat_hbm, a_out, b_out, ea_out,
          idx_r, idx_c, a_buf, b_buf, eat_buf, ea_buf, sem_a, sem_b, sem_e):
        c = lax.axis_index("c")
        s = lax.axis_index("s")
        wid = c * NS + s
        jidx = lax.broadcasted_iota(jnp.int32, (16,), 0)
        for g in range(NG2):
            cbase = wid * NCHUNK + g * KC2
            ebase = wid * EW + g * GE2
            pltpu.sync_copy(rowm_hbm.at[pl.ds(cbase, KC2)], idx_r)
            pltpu.sync_copy(colm_hbm.at[pl.ds(cbase, KC2)], idx_c)
            for j in range(DE):
                pltpu.async_copy(eat_hbm.at[j, pl.ds(ebase, GE2)],
                                 eat_buf.at[j], sem_e)

            def fire(j, carry):
                pltpu.async_copy(xa_hbm.at[idx_r.at[j]],
                                 a_buf.at[pl.ds(j * CL, CL)], sem_a)
                pltpu.async_copy(xb_hbm.at[idx_c.at[j]],
                                 b_buf.at[pl.ds(j * CL, CL)], sem_b)
                return carry

            lax.fori_loop(0, KC2, fire, 0)
            pltpu.make_async_copy(eat_hbm.at[:, pl.ds(0, GE2)], eat_buf,
                                  sem_e).wait()

            def tbody(i, carry):
                base = i * 8
                for t in range(8):
                    ev = jnp.broadcast_to(base + t, (16,)).astype(jnp.int32)
                    v = plsc.load_gather(eat_buf, [jidx, ev])
                    ea_buf[base + t, :] = v
                return carry

            lax.fori_loop(0, GE2 // 8, tbody, 0)
            pltpu.make_async_copy(a_out.at[pl.ds(ebase, GE2)], a_buf, sem_a).wait()
            pltpu.make_async_copy(b_out.at[pl.ds(ebase, GE2)], b_buf, sem_b).wait()
            pltpu.sync_copy(a_buf, a_out.at[pl.ds(ebase, GE2)])
            pltpu.sync_copy(b_buf, b_out.at[pl.ds(ebase, GE2)])
            pltpu.sync_copy(ea_buf, ea_out.at[pl.ds(ebase, GE2)])

    return k(xa, xb, rowm, colm, eat)


EQ = E // 8
BQ = 4000

def _edge_body(a_ref, b_ref, ea_ref, w1e_bd_ref, b1_ref, w2_bd_ref, b2_ref,
               out_ref):
    h1 = (a_ref[...] + b_ref[...]
          + jnp.dot(ea_ref[...], w1e_bd_ref[...], preferred_element_type=_f32)
          + b1_ref[...])
    h1 = jnp.maximum(h1, 0.0)
    h2 = jnp.dot(h1, w2_bd_ref[...], preferred_element_type=_f32) + b2_ref[...]
    out_ref[...] = jnp.maximum(h2, 0.0)


def _tc_edge(a_q, b_q, edge_attr, w1e_bd, b1t, w2_bd, b2t):
    blk = lambda i: (i, 0)
    full = lambda i: (0, 0)
    return pl.pallas_call(
        _edge_body,
        grid=(EQ // BQ,),
        in_specs=[
            pl.BlockSpec((BQ, 128), blk),
            pl.BlockSpec((BQ, 128), blk),
            pl.BlockSpec((BQ, 128), blk),
            pl.BlockSpec((128, 128), full),
            pl.BlockSpec((1, 128), full),
            pl.BlockSpec((128, 128), full),
            pl.BlockSpec((1, 128), full),
        ],
        out_specs=pl.BlockSpec((BQ, 128), blk),
        out_shape=jax.ShapeDtypeStruct((EQ, 128), _f32),
    )(a_q, b_q, edge_attr, w1e_bd, b1t, w2_bd, b2t)


def _sc_scatter(e2, rowm, zeros_nd, ones_cl):
    mesh = plsc.VectorSubcoreMesh(core_axis_name="c", subcore_axis_name="s",
                                  num_cores=NC, num_subcores=NS)

    @functools.partial(
        pl.kernel,
        out_type=(jax.ShapeDtypeStruct((NC, N, 128), _f32),
                  jax.ShapeDtypeStruct((NC, N, 128), _f32),
                  jax.ShapeDtypeStruct((DE, E), _f32)),
        mesh=mesh,
        scratch_types=[
            pltpu.VMEM_SHARED((N, DE), _f32),
            pltpu.VMEM_SHARED((N, DE), _f32),
            pltpu.VMEM((KC, CL), jnp.int32),
            pltpu.VMEM((GE, DE), _f32),
            pltpu.VMEM((DE, GE), _f32),
            pltpu.VMEM((CL, DE), _f32),
            pltpu.SemaphoreType.DMA,
            pltpu.SemaphoreType.DMA,
            pltpu.SemaphoreType.DMA,
        ],
        compiler_params=pltpu.CompilerParams(use_tc_tiling_on_sc=False, needs_layout_passes=False),
    )
    def k(e2_hbm, rowm_hbm, zeros_hbm, ones_hbm, sum_out, cnt_out, e2t_out,
          sum_acc, cnt_acc, idx_v, pay_v, pay_t, ones_v, sem_s, sem_c, sem_t):
        c = lax.axis_index("c")
        s = lax.axis_index("s")
        wid = c * NS + s
        jidx = lax.broadcasted_iota(jnp.int32, (16,), 0)

        @pl.when(s == 0)
        def _():
            pltpu.sync_copy(zeros_hbm, sum_acc)
            pltpu.sync_copy(zeros_hbm, cnt_acc)

        pltpu.sync_copy(ones_hbm, ones_v)
        plsc.subcore_barrier()

        for g in range(NG):
            cbase = wid * NCHUNK + g * KC
            ebase = wid * EW + g * GE
            pltpu.sync_copy(rowm_hbm.at[pl.ds(cbase, KC)], idx_v)
            pltpu.sync_copy(e2_hbm.at[pl.ds(ebase, GE)], pay_v)

            def tbody(i, carry):
                base = i * 8
                for t in range(8):
                    ev = jnp.broadcast_to(base + t, (16,)).astype(jnp.int32)
                    v = pay_v[base + t, :]
                    plsc.store_scatter(pay_t, [jidx, ev], v)
                return carry

            lax.fori_loop(0, GE // 8, tbody, 0)
            for j in range(DE):
                pltpu.async_copy(pay_t.at[j],
                                 e2t_out.at[j, pl.ds(ebase, GE)], sem_t)

            def fire(j, carry):
                pltpu.async_copy(pay_v.at[pl.ds(j * CL, CL)],
                                 sum_acc.at[idx_v.at[j]], sem_s, add=True)
                pltpu.async_copy(ones_v,
                                 cnt_acc.at[idx_v.at[j]], sem_c, add=True)
                return carry

            lax.fori_loop(0, KC, fire, 0)
            pltpu.make_async_copy(zeros_hbm.at[pl.ds(0, GE)], pay_v, sem_s).wait()
            pltpu.make_async_copy(zeros_hbm.at[pl.ds(0, GE)], pay_v, sem_c).wait()
            pltpu.make_async_copy(e2t_out.at[:, pl.ds(0, GE)], pay_t, sem_t).wait()

        plsc.subcore_barrier()

        @pl.when(s < NWR)
        def _():
            pltpu.sync_copy(sum_acc.at[pl.ds(s * NR, NR)],
                            sum_out.at[c, pl.ds(s * NR, NR), pl.ds(0, DE)])
            pltpu.sync_copy(cnt_acc.at[pl.ds(s * NR, NR)],
                            cnt_out.at[c, pl.ds(s * NR, NR), pl.ds(0, DE)])

    return k(e2, rowm, zeros_nd, ones_cl)


def _node_body(x_ref, batch_ref, sums_ref, cnts_ref, u_ref,
               wvx_ref, wve_ref, wvu_ref, b1_ref, w2_ref, b2_ref,
               wua_ref, wub_ref, wuc_ref, ub1_ref, uw2_ref, ub2_ref,
               x2_ref, u2_ref,
               enum_acc, eden_acc, vnum_acc, vden_acc):
    i = pl.program_id(0)
    nsteps = pl.num_programs(0)
    gids = lax.broadcasted_iota(jnp.int32, (1, B), 1)
    oh = (batch_ref[...] == gids).astype(_f32)
    sum_blk = sums_ref[0, :, :DE] + sums_ref[1, :, :DE]
    cnt_blk = cnts_ref[0, :, :DE] + cnts_ref[1, :, :DE]
    e_aggr = sum_blk / jnp.maximum(cnt_blk, 1.0)

    uvw = jnp.dot(u_ref[...], wvu_ref[...], preferred_element_type=_f32)
    h = (jnp.dot(x_ref[...], wvx_ref[...], preferred_element_type=_f32)
         + jnp.dot(e_aggr, wve_ref[...], preferred_element_type=_f32)
         + jnp.dot(oh, uvw, preferred_element_type=_f32)
         + b1_ref[...])
    h = jnp.maximum(h, 0.0)
    x2 = jnp.dot(h, w2_ref[...], preferred_element_type=_f32) + b2_ref[...]
    x2 = jnp.maximum(x2, 0.0)
    x2_ref[...] = x2

    @pl.when(i == 0)
    def _():
        enum_acc[...] = jnp.zeros((B, DE), _f32)
        eden_acc[...] = jnp.zeros((B, DE), _f32)
        vnum_acc[...] = jnp.zeros((B, DV), _f32)
        vden_acc[...] = jnp.zeros((B, DE), _f32)

    contract0 = (((0,), (0,)), ((), ()))
    enum_acc[...] += lax.dot_general(oh, sum_blk, contract0,
                                     preferred_element_type=_f32)
    eden_acc[...] += lax.dot_general(oh, cnt_blk, contract0,
                                     preferred_element_type=_f32)
    vnum_acc[...] += lax.dot_general(oh, x2, contract0,
                                     preferred_element_type=_f32)
    vden_acc[...] += lax.dot_general(oh, jnp.ones((BN, DE), _f32), contract0,
                                     preferred_element_type=_f32)

    @pl.when(i == nsteps - 1)
    def _():
        e_mean = enum_acc[...] / jnp.maximum(eden_acc[...], 1.0)
        v_mean = vnum_acc[...] / jnp.maximum(vden_acc[...][:, :1], 1.0)
        hu = (jnp.dot(u_ref[...], wua_ref[...], preferred_element_type=_f32)
              + jnp.dot(e_mean, wub_ref[...], preferred_element_type=_f32)
              + jnp.dot(v_mean, wuc_ref[...], preferred_element_type=_f32)
              + ub1_ref[...])
        hu = jnp.maximum(hu, 0.0)
        u2 = jnp.dot(hu, uw2_ref[...], preferred_element_type=_f32) + ub2_ref[...]
        u2_ref[...] = jnp.maximum(u2, 0.0)


def _tc_node(x, batch2d, sums, cnts, u,
             wvx, wve, wvu, pv_b1, pv_W2, pv_b2,
             wua, wub, wuc, pu_b1, pu_W2, pu_b2):
    full = lambda i: (0, 0)
    return pl.pallas_call(
        _node_body,
        grid=(N // BN,),
        in_specs=[
            pl.BlockSpec((BN, DV), lambda i: (i, 0)),
            pl.BlockSpec((BN, 1), lambda i: (i, 0)),
            pl.BlockSpec((NC, BN, 128), lambda i: (0, i, 0)),
            pl.BlockSpec((NC, BN, 128), lambda i: (0, i, 0)),
            pl.BlockSpec((B, DU), full),
            pl.BlockSpec((DV, DV), full),
            pl.BlockSpec((DE, DV), full),
            pl.BlockSpec((DU, DV), full),
            pl.BlockSpec((1, DV), full),
            pl.BlockSpec((DV, DV), full),
            pl.BlockSpec((1, DV), full),
            pl.BlockSpec((DU, DU), full),
            pl.BlockSpec((DE, DU), full),
            pl.BlockSpec((DV, DU), full),
            pl.BlockSpec((1, DU), full),
            pl.BlockSpec((DU, DU), full),
            pl.BlockSpec((1, DU), full),
        ],
        out_specs=[
            pl.BlockSpec((BN, DV), lambda i: (i, 0)),
            pl.BlockSpec((B, DU), full),
        ],
        out_shape=[
            jax.ShapeDtypeStruct((N, DV), _f32),
            jax.ShapeDtypeStruct((B, DU), _f32),
        ],
        scratch_shapes=[
            pltpu.VMEM((B, DE), _f32),
            pltpu.VMEM((B, DE), _f32),
            pltpu.VMEM((B, DV), _f32),
            pltpu.VMEM((B, DE), _f32),
        ],
    )(x, batch2d, sums, cnts, u,
      wvx, wve, wvu, pv_b1, pv_W2, pv_b2,
      wua, wub, wuc, pu_b1, pu_W2, pu_b2)


def kernel(x, edge_index, edge_attr, u, batch,
           pe_W1, pe_b1, pe_W2, pe_b2,
           pv_W1, pv_b1, pv_W2, pv_b2,
           pu_W1, pu_b1, pu_W2, pu_b2):
    row = edge_index[0].astype(jnp.int32)
    col = edge_index[1].astype(jnp.int32)
    batch2d = batch.astype(jnp.int32).reshape(N, 1)
    rowm = row.reshape(NW * NCHUNK, CL)
    colm = col.reshape(NW * NCHUNK, CL)

    w1a = pe_W1[:DV]
    w1b = pe_W1[DV:2 * DV]
    w1e = pe_W1[2 * DV:2 * DV + DE]
    w1u = pe_W1[2 * DV + DE:]

    xa, xb = _tc_prep(x, batch2d, u, w1a, w1b, w1u)
    a_g, b_g, ea_lin = _sc_gather(xa, xb, rowm, colm, edge_attr.T)
    eye8 = jnp.eye(8, dtype=_f32)
    e2_q = _tc_edge(a_g.reshape(EQ, 128), b_g.reshape(EQ, 128),
                    ea_lin.reshape(EQ, 128),
                    jnp.kron(eye8, w1e), jnp.tile(pe_b1, 8).reshape(1, 128),
                    jnp.kron(eye8, pe_W2), jnp.tile(pe_b2, 8).reshape(1, 128))
    e2 = e2_q.reshape(E, DE)
    zeros_nd = jnp.zeros((N, DE), _f32)
    ones_cl = jnp.ones((CL, DE), _f32)
    sums, cnts, e2t = _sc_scatter(e2, rowm, zeros_nd, ones_cl)

    wvx = pv_W1[:DV]
    wve = pv_W1[DV:DV + DE]
    wvu = pv_W1[DV + DE:]
    wua = pu_W1[:DU]
    wub = pu_W1[DU:DU + DE]
    wuc = pu_W1[DU + DE:]
    x2, u2 = _tc_node(x, batch2d, sums, cnts, u,
                      wvx, wve, wvu, pv_b1.reshape(1, DV), pv_W2,
                      pv_b2.reshape(1, DV),
                      wua, wub, wuc, pu_b1.reshape(1, DU), pu_W2,
                      pu_b2.reshape(1, DU))
    return (x2, e2t.T, u2)


# speedup vs baseline: 1.0489x; 1.0489x over previous
import functools

import jax
import jax.numpy as jnp
from jax import lax
from jax.experimental import pallas as pl
from jax.experimental.pallas import tpu as pltpu
from jax.experimental.pallas import tpu_sc as plsc

N = 10000
E = 320000
B = 128
DV = 128
DE = 16
DU = 64

NC = 2
NS = 16
NW = NC * NS
EW = E // NW
CL = 125
NCHUNK = EW // CL
KC = 16
NG = NCHUNK // KC
GE = KC * CL
NWR = 10
NR = N // NWR
KC2 = 8
GE2 = KC2 * CL
NG2 = NCHUNK // KC2

BN = 2000
BE = 8000

_f32 = jnp.float32


def _prep_body(x_ref, batch_ref, u_ref, w1a_ref, w1b_ref, w1u_ref,
               xa_ref, xb_ref):
    ug = jnp.dot(u_ref[...], w1u_ref[...], preferred_element_type=_f32)
    gids = lax.broadcasted_iota(jnp.int32, (1, B), 1)
    oh = (batch_ref[...] == gids).astype(_f32)
    xa_ref[...] = (jnp.dot(x_ref[...], w1a_ref[...], preferred_element_type=_f32)
                   + jnp.dot(oh, ug, preferred_element_type=_f32))
    xb_ref[...] = jnp.dot(x_ref[...], w1b_ref[...], preferred_element_type=_f32)


def _tc_prep(x, batch2d, u, w1a, w1b, w1u):
    return pl.pallas_call(
        _prep_body,
        grid=(N // BN,),
        in_specs=[
            pl.BlockSpec((BN, DV), lambda i: (i, 0)),
            pl.BlockSpec((BN, 1), lambda i: (i, 0)),
            pl.BlockSpec((B, DU), lambda i: (0, 0)),
            pl.BlockSpec((DV, DE), lambda i: (0, 0)),
            pl.BlockSpec((DV, DE), lambda i: (0, 0)),
            pl.BlockSpec((DU, DE), lambda i: (0, 0)),
        ],
        out_specs=[
            pl.BlockSpec((BN, DE), lambda i: (i, 0)),
            pl.BlockSpec((BN, DE), lambda i: (i, 0)),
        ],
        out_shape=[
            jax.ShapeDtypeStruct((N, DE), _f32),
            jax.ShapeDtypeStruct((N, DE), _f32),
        ],
    )(x, batch2d, u, w1a, w1b, w1u)


def _sc_gather(xa, xb, rowm, colm, eat):
    mesh = plsc.VectorSubcoreMesh(core_axis_name="c", subcore_axis_name="s",
                                  num_cores=NC, num_subcores=NS)

    @functools.partial(
        pl.kernel,
        out_type=(jax.ShapeDtypeStruct((E, DE), _f32),
                  jax.ShapeDtypeStruct((E, DE), _f32),
                  jax.ShapeDtypeStruct((E, DE), _f32)),
        mesh=mesh,
        scratch_types=[
            pltpu.VMEM((KC2, CL), jnp.int32),
            pltpu.VMEM((KC2, CL), jnp.int32),
            pltpu.VMEM((GE2, DE), _f32),
            pltpu.VMEM((GE2, DE), _f32),
            pltpu.VMEM((DE, GE2), _f32),
            pltpu.VMEM((GE2, DE), _f32),
            pltpu.SemaphoreType.DMA,
            pltpu.SemaphoreType.DMA,
            pltpu.SemaphoreType.DMA,
            pltpu.SemaphoreType.DMA,
        ],
        compiler_params=pltpu.CompilerParams(use_tc_tiling_on_sc=False, needs_layout_passes=False),
    )
    def k(xa_hbm, xb_hbm, rowm_hbm, colm_hbm, eat_hbm, a_out, b_out, ea_out,
          idx_r, idx_c, a_buf, b_buf, eat_buf, ea_buf,
          sem_a, sem_b, sem_e, sem_o):
        c = lax.axis_index("c")
        s = lax.axis_index("s")
        wid = c * NS + s
        jidx = lax.broadcasted_iota(jnp.int32, (16,), 0)
        for g in range(NG2):
            cbase = wid * NCHUNK + g * KC2
            ebase = wid * EW + g * GE2
            pltpu.sync_copy(rowm_hbm.at[pl.ds(cbase, KC2)], idx_r)
            pltpu.sync_copy(colm_hbm.at[pl.ds(cbase, KC2)], idx_c)
            for j in range(DE):
                pltpu.async_copy(eat_hbm.at[j, pl.ds(ebase, GE2)],
                                 eat_buf.at[j], sem_e)
            if g > 0:
                pb = wid * EW + (g - 1) * GE2
                pltpu.make_async_copy(a_out.at[pl.ds(pb, GE2)], a_buf, sem_o).wait()
                pltpu.make_async_copy(b_out.at[pl.ds(pb, GE2)], b_buf, sem_o).wait()
                pltpu.make_async_copy(a_out.at[pl.ds(pb, GE2)], ea_buf, sem_o).wait()

            def fire(j, carry):
                pltpu.async_copy(xa_hbm.at[idx_r.at[j]],
                                 a_buf.at[pl.ds(j * CL, CL)], sem_a)
                pltpu.async_copy(xb_hbm.at[idx_c.at[j]],
                                 b_buf.at[pl.ds(j * CL, CL)], sem_b)
                return carry

            lax.fori_loop(0, KC2, fire, 0)
            pltpu.make_async_copy(eat_hbm.at[:, pl.ds(0, GE2)], eat_buf,
                                  sem_e).wait()

            def tbody(e, carry):
                ev = jnp.broadcast_to(e, (16,)).astype(jnp.int32)
                v = plsc.load_gather(eat_buf, [jidx, ev])
                ea_buf[e, :] = v
                return carry

            lax.fori_loop(0, GE2, tbody, 0)
            pltpu.make_async_copy(a_out.at[pl.ds(ebase, GE2)], a_buf, sem_a).wait()
            pltpu.make_async_copy(b_out.at[pl.ds(ebase, GE2)], b_buf, sem_b).wait()
            pltpu.async_copy(a_buf, a_out.at[pl.ds(ebase, GE2)], sem_o)
            pltpu.async_copy(b_buf, b_out.at[pl.ds(ebase, GE2)], sem_o)
            pltpu.async_copy(ea_buf, ea_out.at[pl.ds(ebase, GE2)], sem_o)
        fb = wid * EW + (NG2 - 1) * GE2
        pltpu.make_async_copy(a_out.at[pl.ds(fb, GE2)], a_buf, sem_o).wait()
        pltpu.make_async_copy(b_out.at[pl.ds(fb, GE2)], b_buf, sem_o).wait()
        pltpu.make_async_copy(a_out.at[pl.ds(fb, GE2)], ea_buf, sem_o).wait()

    return k(xa, xb, rowm, colm, eat)


EQ = E // 8
BQ = 4000

def _edge_body(a_ref, b_ref, ea_ref, w1e_bd_ref, b1_ref, w2_bd_ref, b2_ref,
               out_ref):
    h1 = (a_ref[...] + b_ref[...]
          + jnp.dot(ea_ref[...], w1e_bd_ref[...], preferred_element_type=_f32)
          + b1_ref[...])
    h1 = jnp.maximum(h1, 0.0)
    h2 = jnp.dot(h1, w2_bd_ref[...], preferred_element_type=_f32) + b2_ref[...]
    out_ref[...] = jnp.maximum(h2, 0.0)


def _tc_edge(a_q, b_q, edge_attr, w1e_bd, b1t, w2_bd, b2t):
    blk = lambda i: (i, 0)
    full = lambda i: (0, 0)
    return pl.pallas_call(
        _edge_body,
        grid=(EQ // BQ,),
        in_specs=[
            pl.BlockSpec((BQ, 128), blk),
            pl.BlockSpec((BQ, 128), blk),
            pl.BlockSpec((BQ, 128), blk),
            pl.BlockSpec((128, 128), full),
            pl.BlockSpec((1, 128), full),
            pl.BlockSpec((128, 128), full),
            pl.BlockSpec((1, 128), full),
        ],
        out_specs=pl.BlockSpec((BQ, 128), blk),
        out_shape=jax.ShapeDtypeStruct((EQ, 128), _f32),
    )(a_q, b_q, edge_attr, w1e_bd, b1t, w2_bd, b2t)


def _sc_scatter(e2, rowm, zeros_nd, ones_cl):
    mesh = plsc.VectorSubcoreMesh(core_axis_name="c", subcore_axis_name="s",
                                  num_cores=NC, num_subcores=NS)

    @functools.partial(
        pl.kernel,
        out_type=(jax.ShapeDtypeStruct((NC, N, 128), _f32),
                  jax.ShapeDtypeStruct((NC, N, 128), _f32),
                  jax.ShapeDtypeStruct((DE, E), _f32)),
        mesh=mesh,
        scratch_types=[
            pltpu.VMEM_SHARED((N, DE), _f32),
            pltpu.VMEM_SHARED((N, DE), _f32),
            pltpu.VMEM((KC, CL), jnp.int32),
            pltpu.VMEM((GE, DE), _f32),
            pltpu.VMEM((DE, GE), _f32),
            pltpu.VMEM((CL, DE), _f32),
            pltpu.SemaphoreType.DMA,
            pltpu.SemaphoreType.DMA,
            pltpu.SemaphoreType.DMA,
        ],
        compiler_params=pltpu.CompilerParams(use_tc_tiling_on_sc=False, needs_layout_passes=False),
    )
    def k(e2_hbm, rowm_hbm, zeros_hbm, ones_hbm, sum_out, cnt_out, e2t_out,
          sum_acc, cnt_acc, idx_v, pay_v, pay_t, ones_v, sem_s, sem_c, sem_t):
        c = lax.axis_index("c")
        s = lax.axis_index("s")
        wid = c * NS + s
        jidx = lax.broadcasted_iota(jnp.int32, (16,), 0)

        @pl.when(s == 0)
        def _():
            pltpu.sync_copy(zeros_hbm, sum_acc)
            pltpu.sync_copy(zeros_hbm, cnt_acc)

        pltpu.sync_copy(ones_hbm, ones_v)
        plsc.subcore_barrier()

        for g in range(NG):
            cbase = wid * NCHUNK + g * KC
            ebase = wid * EW + g * GE
            pltpu.sync_copy(rowm_hbm.at[pl.ds(cbase, KC)], idx_v)
            pltpu.sync_copy(e2_hbm.at[pl.ds(ebase, GE)], pay_v)

            def tbody(e, carry):
                ev = jnp.broadcast_to(e, (16,)).astype(jnp.int32)
                v = pay_v[e, :]
                plsc.store_scatter(pay_t, [jidx, ev], v)
                return carry

            lax.fori_loop(0, GE, tbody, 0)
            for j in range(DE):
                pltpu.async_copy(pay_t.at[j],
                                 e2t_out.at[j, pl.ds(ebase, GE)], sem_t)

            def fire(j, carry):
                pltpu.async_copy(pay_v.at[pl.ds(j * CL, CL)],
                                 sum_acc.at[idx_v.at[j]], sem_s, add=True)
                pltpu.async_copy(ones_v,
                                 cnt_acc.at[idx_v.at[j]], sem_c, add=True)
                return carry

            lax.fori_loop(0, KC, fire, 0)
            pltpu.make_async_copy(zeros_hbm.at[pl.ds(0, GE)], pay_v, sem_s).wait()
            pltpu.make_async_copy(zeros_hbm.at[pl.ds(0, GE)], pay_v, sem_c).wait()
            pltpu.make_async_copy(e2t_out.at[:, pl.ds(0, GE)], pay_t, sem_t).wait()

        plsc.subcore_barrier()

        @pl.when(s < NWR)
        def _():
            pltpu.sync_copy(sum_acc.at[pl.ds(s * NR, NR)],
                            sum_out.at[c, pl.ds(s * NR, NR), pl.ds(0, DE)])
            pltpu.sync_copy(cnt_acc.at[pl.ds(s * NR, NR)],
                            cnt_out.at[c, pl.ds(s * NR, NR), pl.ds(0, DE)])

    return k(e2, rowm, zeros_nd, ones_cl)


def _node_body(x_ref, batch_ref, sums_ref, cnts_ref, u_ref,
               wvx_ref, wve_ref, wvu_ref, b1_ref, w2_ref, b2_ref,
               wua_ref, wub_ref, wuc_ref, ub1_ref, uw2_ref, ub2_ref,
               x2_ref, u2_ref,
               enum_acc, eden_acc, vnum_acc, vden_acc):
    i = pl.program_id(0)
    nsteps = pl.num_programs(0)
    gids = lax.broadcasted_iota(jnp.int32, (1, B), 1)
    oh = (batch_ref[...] == gids).astype(_f32)
    sum_blk = sums_ref[0, :, :DE] + sums_ref[1, :, :DE]
    cnt_blk = cnts_ref[0, :, :DE] + cnts_ref[1, :, :DE]
    e_aggr = sum_blk / jnp.maximum(cnt_blk, 1.0)

    uvw = jnp.dot(u_ref[...], wvu_ref[...], preferred_element_type=_f32)
    h = (jnp.dot(x_ref[...], wvx_ref[...], preferred_element_type=_f32)
         + jnp.dot(e_aggr, wve_ref[...], preferred_element_type=_f32)
         + jnp.dot(oh, uvw, preferred_element_type=_f32)
         + b1_ref[...])
    h = jnp.maximum(h, 0.0)
    x2 = jnp.dot(h, w2_ref[...], preferred_element_type=_f32) + b2_ref[...]
    x2 = jnp.maximum(x2, 0.0)
    x2_ref[...] = x2

    @pl.when(i == 0)
    def _():
        enum_acc[...] = jnp.zeros((B, DE), _f32)
        eden_acc[...] = jnp.zeros((B, DE), _f32)
        vnum_acc[...] = jnp.zeros((B, DV), _f32)
        vden_acc[...] = jnp.zeros((B, DE), _f32)

    contract0 = (((0,), (0,)), ((), ()))
    enum_acc[...] += lax.dot_general(oh, sum_blk, contract0,
                                     preferred_element_type=_f32)
    eden_acc[...] += lax.dot_general(oh, cnt_blk, contract0,
                                     preferred_element_type=_f32)
    vnum_acc[...] += lax.dot_general(oh, x2, contract0,
                                     preferred_element_type=_f32)
    vden_acc[...] += lax.dot_general(oh, jnp.ones((BN, DE), _f32), contract0,
                                     preferred_element_type=_f32)

    @pl.when(i == nsteps - 1)
    def _():
        e_mean = enum_acc[...] / jnp.maximum(eden_acc[...], 1.0)
        v_mean = vnum_acc[...] / jnp.maximum(vden_acc[...][:, :1], 1.0)
        hu = (jnp.dot(u_ref[...], wua_ref[...], preferred_element_type=_f32)
              + jnp.dot(e_mean, wub_ref[...], preferred_element_type=_f32)
              + jnp.dot(v_mean, wuc_ref[...], preferred_element_type=_f32)
              + ub1_ref[...])
        hu = jnp.maximum(hu, 0.0)
        u2 = jnp.dot(hu, uw2_ref[...], preferred_element_type=_f32) + ub2_ref[...]
        u2_ref[...] = jnp.maximum(u2, 0.0)


def _tc_node(x, batch2d, sums, cnts, u,
             wvx, wve, wvu, pv_b1, pv_W2, pv_b2,
             wua, wub, wuc, pu_b1, pu_W2, pu_b2):
    full = lambda i: (0, 0)
    return pl.pallas_call(
        _node_body,
        grid=(N // BN,),
        in_specs=[
            pl.BlockSpec((BN, DV), lambda i: (i, 0)),
            pl.BlockSpec((BN, 1), lambda i: (i, 0)),
            pl.BlockSpec((NC, BN, 128), lambda i: (0, i, 0)),
            pl.BlockSpec((NC, BN, 128), lambda i: (0, i, 0)),
            pl.BlockSpec((B, DU), full),
            pl.BlockSpec((DV, DV), full),
            pl.BlockSpec((DE, DV), full),
            pl.BlockSpec((DU, DV), full),
            pl.BlockSpec((1, DV), full),
            pl.BlockSpec((DV, DV), full),
            pl.BlockSpec((1, DV), full),
            pl.BlockSpec((DU, DU), full),
            pl.BlockSpec((DE, DU), full),
            pl.BlockSpec((DV, DU), full),
            pl.BlockSpec((1, DU), full),
            pl.BlockSpec((DU, DU), full),
            pl.BlockSpec((1, DU), full),
        ],
        out_specs=[
            pl.BlockSpec((BN, DV), lambda i: (i, 0)),
            pl.BlockSpec((B, DU), full),
        ],
        out_shape=[
            jax.ShapeDtypeStruct((N, DV), _f32),
            jax.ShapeDtypeStruct((B, DU), _f32),
        ],
        scratch_shapes=[
            pltpu.VMEM((B, DE), _f32),
            pltpu.VMEM((B, DE), _f32),
            pltpu.VMEM((B, DV), _f32),
            pltpu.VMEM((B, DE), _f32),
        ],
    )(x, batch2d, sums, cnts, u,
      wvx, wve, wvu, pv_b1, pv_W2, pv_b2,
      wua, wub, wuc, pu_b1, pu_W2, pu_b2)


def kernel(x, edge_index, edge_attr, u, batch,
           pe_W1, pe_b1, pe_W2, pe_b2,
           pv_W1, pv_b1, pv_W2, pv_b2,
           pu_W1, pu_b1, pu_W2, pu_b2):
    row = edge_index[0].astype(jnp.int32)
    col = edge_index[1].astype(jnp.int32)
    batch2d = batch.astype(jnp.int32).reshape(N, 1)
    rowm = row.reshape(NW * NCHUNK, CL)
    colm = col.reshape(NW * NCHUNK, CL)

    w1a = pe_W1[:DV]
    w1b = pe_W1[DV:2 * DV]
    w1e = pe_W1[2 * DV:2 * DV + DE]
    w1u = pe_W1[2 * DV + DE:]

    xa, xb = _tc_prep(x, batch2d, u, w1a, w1b, w1u)
    a_g, b_g, ea_lin = _sc_gather(xa, xb, rowm, colm, edge_attr.T)
    eye8 = jnp.eye(8, dtype=_f32)
    e2_q = _tc_edge(a_g.reshape(EQ, 128), b_g.reshape(EQ, 128),
                    ea_lin.reshape(EQ, 128),
                    jnp.kron(eye8, w1e), jnp.tile(pe_b1, 8).reshape(1, 128),
                    jnp.kron(eye8, pe_W2), jnp.tile(pe_b2, 8).reshape(1, 128))
    e2 = e2_q.reshape(E, DE)
    zeros_nd = jnp.zeros((N, DE), _f32)
    ones_cl = jnp.ones((CL, DE), _f32)
    sums, cnts, e2t = _sc_scatter(e2, rowm, zeros_nd, ones_cl)

    wvx = pv_W1[:DV]
    wve = pv_W1[DV:DV + DE]
    wvu = pv_W1[DV + DE:]
    wua = pu_W1[:DU]
    wub = pu_W1[DU:DU + DE]
    wuc = pu_W1[DU + DE:]
    x2, u2 = _tc_node(x, batch2d, sums, cnts, u,
                      wvx, wve, wvu, pv_b1.reshape(1, DV), pv_W2,
                      pv_b2.reshape(1, DV),
                      wua, wub, wuc, pu_b1.reshape(1, DU), pu_W2,
                      pu_b2.reshape(1, DU))
    return (x2, e2t.T, u2)


# speedup vs baseline: 1.1113x; 1.0595x over previous
import functools

import jax
import jax.numpy as jnp
from jax import lax
from jax.experimental import pallas as pl
from jax.experimental.pallas import tpu as pltpu
from jax.experimental.pallas import tpu_sc as plsc

N = 10000
E = 320000
B = 128
DV = 128
DE = 16
DU = 64

NC = 2
NS = 16
NW = NC * NS
EW = E // NW
CL = 125
NCHUNK = EW // CL
KC = 16
NG = NCHUNK // KC
GE = KC * CL
NWR = 10
NR = N // NWR
KC2 = 8
GE2 = KC2 * CL
NG2 = NCHUNK // KC2

BN = 2000
BE = 8000

_f32 = jnp.float32


def _prep_body(x_ref, batch_ref, u_ref, w1a_ref, w1b_ref, w1u_ref,
               xa_ref, xb_ref):
    ug = jnp.dot(u_ref[...], w1u_ref[...], preferred_element_type=_f32)
    gids = lax.broadcasted_iota(jnp.int32, (1, B), 1)
    oh = (batch_ref[...] == gids).astype(_f32)
    xa_ref[...] = (jnp.dot(x_ref[...], w1a_ref[...], preferred_element_type=_f32)
                   + jnp.dot(oh, ug, preferred_element_type=_f32))
    xb_ref[...] = jnp.dot(x_ref[...], w1b_ref[...], preferred_element_type=_f32)


def _tc_prep(x, batch2d, u, w1a, w1b, w1u):
    return pl.pallas_call(
        _prep_body,
        grid=(N // BN,),
        in_specs=[
            pl.BlockSpec((BN, DV), lambda i: (i, 0)),
            pl.BlockSpec((BN, 1), lambda i: (i, 0)),
            pl.BlockSpec((B, DU), lambda i: (0, 0)),
            pl.BlockSpec((DV, DE), lambda i: (0, 0)),
            pl.BlockSpec((DV, DE), lambda i: (0, 0)),
            pl.BlockSpec((DU, DE), lambda i: (0, 0)),
        ],
        out_specs=[
            pl.BlockSpec((BN, DE), lambda i: (i, 0)),
            pl.BlockSpec((BN, DE), lambda i: (i, 0)),
        ],
        out_shape=[
            jax.ShapeDtypeStruct((N, DE), _f32),
            jax.ShapeDtypeStruct((N, DE), _f32),
        ],
    )(x, batch2d, u, w1a, w1b, w1u)


def _sc_gather(xa, xb, rowm, colm, eat):
    mesh = plsc.VectorSubcoreMesh(core_axis_name="c", subcore_axis_name="s",
                                  num_cores=NC, num_subcores=NS)

    @functools.partial(
        pl.kernel,
        out_type=(jax.ShapeDtypeStruct((E, DE), _f32),
                  jax.ShapeDtypeStruct((E, DE), _f32),
                  jax.ShapeDtypeStruct((E, DE), _f32)),
        mesh=mesh,
        scratch_types=[
            pltpu.VMEM((2, KC2, CL), jnp.int32),
            pltpu.VMEM((2, KC2, CL), jnp.int32),
            pltpu.VMEM((GE2, DE), _f32),
            pltpu.VMEM((GE2, DE), _f32),
            pltpu.VMEM((DE, GE2), _f32),
            pltpu.VMEM((GE2, DE), _f32),
            pltpu.SemaphoreType.DMA,
            pltpu.SemaphoreType.DMA,
            pltpu.SemaphoreType.DMA,
            pltpu.SemaphoreType.DMA,
            pltpu.SemaphoreType.DMA,
        ],
        compiler_params=pltpu.CompilerParams(use_tc_tiling_on_sc=False, needs_layout_passes=False),
    )
    def k(xa_hbm, xb_hbm, rowm_hbm, colm_hbm, eat_hbm, a_out, b_out, ea_out,
          idx_r, idx_c, a_buf, b_buf, eat_buf, ea_buf,
          sem_a, sem_b, sem_e, sem_o, sem_i):
        c = lax.axis_index("c")
        s = lax.axis_index("s")
        wid = c * NS + s
        jidx = lax.broadcasted_iota(jnp.int32, (16,), 0)
        pltpu.sync_copy(rowm_hbm.at[pl.ds(wid * NCHUNK, KC2)], idx_r.at[0])
        pltpu.sync_copy(colm_hbm.at[pl.ds(wid * NCHUNK, KC2)], idx_c.at[0])
        for g in range(NG2):
            ebase = wid * EW + g * GE2
            for j in range(DE):
                pltpu.async_copy(eat_hbm.at[j, pl.ds(ebase, GE2)],
                                 eat_buf.at[j], sem_e)
            if g > 0:
                pb = wid * EW + (g - 1) * GE2
                pltpu.make_async_copy(a_out.at[pl.ds(pb, GE2)], a_buf, sem_o).wait()
                pltpu.make_async_copy(b_out.at[pl.ds(pb, GE2)], b_buf, sem_o).wait()
                pltpu.make_async_copy(a_out.at[pl.ds(pb, GE2)], ea_buf, sem_o).wait()
                pltpu.make_async_copy(rowm_hbm.at[pl.ds(0, KC2)],
                                      idx_r.at[g % 2], sem_i).wait()
                pltpu.make_async_copy(rowm_hbm.at[pl.ds(0, KC2)],
                                      idx_c.at[g % 2], sem_i).wait()
            if g + 1 < NG2:
                nbase = wid * NCHUNK + (g + 1) * KC2
                pltpu.async_copy(rowm_hbm.at[pl.ds(nbase, KC2)],
                                 idx_r.at[(g + 1) % 2], sem_i)
                pltpu.async_copy(colm_hbm.at[pl.ds(nbase, KC2)],
                                 idx_c.at[(g + 1) % 2], sem_i)

            ir = idx_r.at[g % 2]
            ic = idx_c.at[g % 2]

            def fire(j, carry):
                pltpu.async_copy(xa_hbm.at[ir.at[j]],
                                 a_buf.at[pl.ds(j * CL, CL)], sem_a)
                pltpu.async_copy(xb_hbm.at[ic.at[j]],
                                 b_buf.at[pl.ds(j * CL, CL)], sem_b)
                return carry

            lax.fori_loop(0, KC2, fire, 0)
            pltpu.make_async_copy(eat_hbm.at[:, pl.ds(0, GE2)], eat_buf,
                                  sem_e).wait()

            def tbody(e, carry):
                ev = jnp.broadcast_to(e, (16,)).astype(jnp.int32)
                v = plsc.load_gather(eat_buf, [jidx, ev])
                ea_buf[e, :] = v
                return carry

            lax.fori_loop(0, GE2, tbody, 0)
            pltpu.make_async_copy(a_out.at[pl.ds(ebase, GE2)], a_buf, sem_a).wait()
            pltpu.make_async_copy(b_out.at[pl.ds(ebase, GE2)], b_buf, sem_b).wait()
            pltpu.async_copy(a_buf, a_out.at[pl.ds(ebase, GE2)], sem_o)
            pltpu.async_copy(b_buf, b_out.at[pl.ds(ebase, GE2)], sem_o)
            pltpu.async_copy(ea_buf, ea_out.at[pl.ds(ebase, GE2)], sem_o)
        fb = wid * EW + (NG2 - 1) * GE2
        pltpu.make_async_copy(a_out.at[pl.ds(fb, GE2)], a_buf, sem_o).wait()
        pltpu.make_async_copy(b_out.at[pl.ds(fb, GE2)], b_buf, sem_o).wait()
        pltpu.make_async_copy(a_out.at[pl.ds(fb, GE2)], ea_buf, sem_o).wait()

    return k(xa, xb, rowm, colm, eat)


EQ = E // 8
BQ = 4000

def _edge_body(a_ref, b_ref, ea_ref, w1e_bd_ref, b1_ref, w2_bd_ref, b2_ref,
               out_ref):
    h1 = (a_ref[...] + b_ref[...]
          + jnp.dot(ea_ref[...], w1e_bd_ref[...], preferred_element_type=_f32)
          + b1_ref[...])
    h1 = jnp.maximum(h1, 0.0)
    h2 = jnp.dot(h1, w2_bd_ref[...], preferred_element_type=_f32) + b2_ref[...]
    out_ref[...] = jnp.maximum(h2, 0.0)


def _tc_edge(a_q, b_q, edge_attr, w1e_bd, b1t, w2_bd, b2t):
    blk = lambda i: (i, 0)
    full = lambda i: (0, 0)
    return pl.pallas_call(
        _edge_body,
        grid=(EQ // BQ,),
        in_specs=[
            pl.BlockSpec((BQ, 128), blk),
            pl.BlockSpec((BQ, 128), blk),
            pl.BlockSpec((BQ, 128), blk),
            pl.BlockSpec((128, 128), full),
            pl.BlockSpec((1, 128), full),
            pl.BlockSpec((128, 128), full),
            pl.BlockSpec((1, 128), full),
        ],
        out_specs=pl.BlockSpec((BQ, 128), blk),
        out_shape=jax.ShapeDtypeStruct((EQ, 128), _f32),
    )(a_q, b_q, edge_attr, w1e_bd, b1t, w2_bd, b2t)


def _sc_scatter(e2, rowm, zeros_nd, ones_cl):
    mesh = plsc.VectorSubcoreMesh(core_axis_name="c", subcore_axis_name="s",
                                  num_cores=NC, num_subcores=NS)

    @functools.partial(
        pl.kernel,
        out_type=(jax.ShapeDtypeStruct((NC, N, 128), _f32),
                  jax.ShapeDtypeStruct((NC, N, 128), _f32),
                  jax.ShapeDtypeStruct((DE, E), _f32)),
        mesh=mesh,
        scratch_types=[
            pltpu.VMEM_SHARED((N, DE), _f32),
            pltpu.VMEM_SHARED((N, DE), _f32),
            pltpu.VMEM((KC, CL), jnp.int32),
            pltpu.VMEM((GE, DE), _f32),
            pltpu.VMEM((DE, GE), _f32),
            pltpu.VMEM((CL, DE), _f32),
            pltpu.SemaphoreType.DMA,
            pltpu.SemaphoreType.DMA,
            pltpu.SemaphoreType.DMA,
        ],
        compiler_params=pltpu.CompilerParams(use_tc_tiling_on_sc=False, needs_layout_passes=False),
    )
    def k(e2_hbm, rowm_hbm, zeros_hbm, ones_hbm, sum_out, cnt_out, e2t_out,
          sum_acc, cnt_acc, idx_v, pay_v, pay_t, ones_v, sem_s, sem_c, sem_t):
        c = lax.axis_index("c")
        s = lax.axis_index("s")
        wid = c * NS + s
        jidx = lax.broadcasted_iota(jnp.int32, (16,), 0)

        @pl.when(s == 0)
        def _():
            pltpu.sync_copy(zeros_hbm, sum_acc)
            pltpu.sync_copy(zeros_hbm, cnt_acc)

        pltpu.sync_copy(ones_hbm, ones_v)
        plsc.subcore_barrier()

        for g in range(NG):
            cbase = wid * NCHUNK + g * KC
            ebase = wid * EW + g * GE
            pltpu.sync_copy(rowm_hbm.at[pl.ds(cbase, KC)], idx_v)
            pltpu.sync_copy(e2_hbm.at[pl.ds(ebase, GE)], pay_v)
            if g > 0:
                pltpu.make_async_copy(e2t_out.at[:, pl.ds(0, GE)], pay_t,
                                      sem_t).wait()

            def tbody(e, carry):
                ev = jnp.broadcast_to(e, (16,)).astype(jnp.int32)
                v = pay_v[e, :]
                plsc.store_scatter(pay_t, [jidx, ev], v)
                return carry

            lax.fori_loop(0, GE, tbody, 0)
            for j in range(DE):
                pltpu.async_copy(pay_t.at[j],
                                 e2t_out.at[j, pl.ds(ebase, GE)], sem_t)

            def fire(j, carry):
                pltpu.async_copy(pay_v.at[pl.ds(j * CL, CL)],
                                 sum_acc.at[idx_v.at[j]], sem_s, add=True)
                pltpu.async_copy(ones_v,
                                 cnt_acc.at[idx_v.at[j]], sem_c, add=True)
                return carry

            lax.fori_loop(0, KC, fire, 0)
            pltpu.make_async_copy(zeros_hbm.at[pl.ds(0, GE)], pay_v, sem_s).wait()
            pltpu.make_async_copy(zeros_hbm.at[pl.ds(0, GE)], pay_v, sem_c).wait()

        pltpu.make_async_copy(e2t_out.at[:, pl.ds(0, GE)], pay_t, sem_t).wait()
        plsc.subcore_barrier()

        @pl.when(s < NWR)
        def _():
            pltpu.sync_copy(sum_acc.at[pl.ds(s * NR, NR)],
                            sum_out.at[c, pl.ds(s * NR, NR), pl.ds(0, DE)])
            pltpu.sync_copy(cnt_acc.at[pl.ds(s * NR, NR)],
                            cnt_out.at[c, pl.ds(s * NR, NR), pl.ds(0, DE)])

    return k(e2, rowm, zeros_nd, ones_cl)


def _node_body(x_ref, batch_ref, sums_ref, cnts_ref, u_ref,
               wvx_ref, wve_ref, wvu_ref, b1_ref, w2_ref, b2_ref,
               wua_ref, wub_ref, wuc_ref, ub1_ref, uw2_ref, ub2_ref,
               x2_ref, u2_ref,
               enum_acc, eden_acc, vnum_acc, vden_acc):
    i = pl.program_id(0)
    nsteps = pl.num_programs(0)
    gids = lax.broadcasted_iota(jnp.int32, (1, B), 1)
    oh = (batch_ref[...] == gids).astype(_f32)
    sum_blk = sums_ref[0, :, :DE] + sums_ref[1, :, :DE]
    cnt_blk = cnts_ref[0, :, :DE] + cnts_ref[1, :, :DE]
    e_aggr = sum_blk / jnp.maximum(cnt_blk, 1.0)

    uvw = jnp.dot(u_ref[...], wvu_ref[...], preferred_element_type=_f32)
    h = (jnp.dot(x_ref[...], wvx_ref[...], preferred_element_type=_f32)
         + jnp.dot(e_aggr, wve_ref[...], preferred_element_type=_f32)
         + jnp.dot(oh, uvw, preferred_element_type=_f32)
         + b1_ref[...])
    h = jnp.maximum(h, 0.0)
    x2 = jnp.dot(h, w2_ref[...], preferred_element_type=_f32) + b2_ref[...]
    x2 = jnp.maximum(x2, 0.0)
    x2_ref[...] = x2

    @pl.when(i == 0)
    def _():
        enum_acc[...] = jnp.zeros((B, DE), _f32)
        eden_acc[...] = jnp.zeros((B, DE), _f32)
        vnum_acc[...] = jnp.zeros((B, DV), _f32)
        vden_acc[...] = jnp.zeros((B, DE), _f32)

    contract0 = (((0,), (0,)), ((), ()))
    enum_acc[...] += lax.dot_general(oh, sum_blk, contract0,
                                     preferred_element_type=_f32)
    eden_acc[...] += lax.dot_general(oh, cnt_blk, contract0,
                                     preferred_element_type=_f32)
    vnum_acc[...] += lax.dot_general(oh, x2, contract0,
                                     preferred_element_type=_f32)
    vden_acc[...] += lax.dot_general(oh, jnp.ones((BN, DE), _f32), contract0,
                                     preferred_element_type=_f32)

    @pl.when(i == nsteps - 1)
    def _():
        e_mean = enum_acc[...] / jnp.maximum(eden_acc[...], 1.0)
        v_mean = vnum_acc[...] / jnp.maximum(vden_acc[...][:, :1], 1.0)
        hu = (jnp.dot(u_ref[...], wua_ref[...], preferred_element_type=_f32)
              + jnp.dot(e_mean, wub_ref[...], preferred_element_type=_f32)
              + jnp.dot(v_mean, wuc_ref[...], preferred_element_type=_f32)
              + ub1_ref[...])
        hu = jnp.maximum(hu, 0.0)
        u2 = jnp.dot(hu, uw2_ref[...], preferred_element_type=_f32) + ub2_ref[...]
        u2_ref[...] = jnp.maximum(u2, 0.0)


def _tc_node(x, batch2d, sums, cnts, u,
             wvx, wve, wvu, pv_b1, pv_W2, pv_b2,
             wua, wub, wuc, pu_b1, pu_W2, pu_b2):
    full = lambda i: (0, 0)
    return pl.pallas_call(
        _node_body,
        grid=(N // BN,),
        in_specs=[
            pl.BlockSpec((BN, DV), lambda i: (i, 0)),
            pl.BlockSpec((BN, 1), lambda i: (i, 0)),
            pl.BlockSpec((NC, BN, 128), lambda i: (0, i, 0)),
            pl.BlockSpec((NC, BN, 128), lambda i: (0, i, 0)),
            pl.BlockSpec((B, DU), full),
            pl.BlockSpec((DV, DV), full),
            pl.BlockSpec((DE, DV), full),
            pl.BlockSpec((DU, DV), full),
            pl.BlockSpec((1, DV), full),
            pl.BlockSpec((DV, DV), full),
            pl.BlockSpec((1, DV), full),
            pl.BlockSpec((DU, DU), full),
            pl.BlockSpec((DE, DU), full),
            pl.BlockSpec((DV, DU), full),
            pl.BlockSpec((1, DU), full),
            pl.BlockSpec((DU, DU), full),
            pl.BlockSpec((1, DU), full),
        ],
        out_specs=[
            pl.BlockSpec((BN, DV), lambda i: (i, 0)),
            pl.BlockSpec((B, DU), full),
        ],
        out_shape=[
            jax.ShapeDtypeStruct((N, DV), _f32),
            jax.ShapeDtypeStruct((B, DU), _f32),
        ],
        scratch_shapes=[
            pltpu.VMEM((B, DE), _f32),
            pltpu.VMEM((B, DE), _f32),
            pltpu.VMEM((B, DV), _f32),
            pltpu.VMEM((B, DE), _f32),
        ],
    )(x, batch2d, sums, cnts, u,
      wvx, wve, wvu, pv_b1, pv_W2, pv_b2,
      wua, wub, wuc, pu_b1, pu_W2, pu_b2)


def kernel(x, edge_index, edge_attr, u, batch,
           pe_W1, pe_b1, pe_W2, pe_b2,
           pv_W1, pv_b1, pv_W2, pv_b2,
           pu_W1, pu_b1, pu_W2, pu_b2):
    row = edge_index[0].astype(jnp.int32)
    col = edge_index[1].astype(jnp.int32)
    batch2d = batch.astype(jnp.int32).reshape(N, 1)
    rowm = row.reshape(NW * NCHUNK, CL)
    colm = col.reshape(NW * NCHUNK, CL)

    w1a = pe_W1[:DV]
    w1b = pe_W1[DV:2 * DV]
    w1e = pe_W1[2 * DV:2 * DV + DE]
    w1u = pe_W1[2 * DV + DE:]

    xa, xb = _tc_prep(x, batch2d, u, w1a, w1b, w1u)
    a_g, b_g, ea_lin = _sc_gather(xa, xb, rowm, colm, edge_attr.T)
    eye8 = jnp.eye(8, dtype=_f32)
    e2_q = _tc_edge(a_g.reshape(EQ, 128), b_g.reshape(EQ, 128),
                    ea_lin.reshape(EQ, 128),
                    jnp.kron(eye8, w1e), jnp.tile(pe_b1, 8).reshape(1, 128),
                    jnp.kron(eye8, pe_W2), jnp.tile(pe_b2, 8).reshape(1, 128))
    e2 = e2_q.reshape(E, DE)
    zeros_nd = jnp.zeros((N, DE), _f32)
    ones_cl = jnp.ones((CL, DE), _f32)
    sums, cnts, e2t = _sc_scatter(e2, rowm, zeros_nd, ones_cl)

    wvx = pv_W1[:DV]
    wve = pv_W1[DV:DV + DE]
    wvu = pv_W1[DV + DE:]
    wua = pu_W1[:DU]
    wub = pu_W1[DU:DU + DE]
    wuc = pu_W1[DU + DE:]
    x2, u2 = _tc_node(x, batch2d, sums, cnts, u,
                      wvx, wve, wvu, pv_b1.reshape(1, DV), pv_W2,
                      pv_b2.reshape(1, DV),
                      wua, wub, wuc, pu_b1.reshape(1, DU), pu_W2,
                      pu_b2.reshape(1, DU))
    return (x2, e2t.T, u2)


# speedup vs baseline: 1.1487x; 1.0336x over previous
import functools

import jax
import jax.numpy as jnp
from jax import lax
from jax.experimental import pallas as pl
from jax.experimental.pallas import tpu as pltpu
from jax.experimental.pallas import tpu_sc as plsc

N = 10000
E = 320000
B = 128
DV = 128
DE = 16
DU = 64

NC = 2
NS = 16
NW = NC * NS
EW = E // NW
CL = 125
NCHUNK = EW // CL
KC = 16
NG = NCHUNK // KC
GE = KC * CL
NWR = 10
NR = N // NWR
KC2 = 8
GE2 = KC2 * CL
NG2 = NCHUNK // KC2

BN = 2000
BE = 8000

_f32 = jnp.float32


def _prep_body(x_ref, batch_ref, u_ref, w1a_ref, w1b_ref, w1u_ref,
               xa_ref, xb_ref):
    ug = jnp.dot(u_ref[...], w1u_ref[...], preferred_element_type=_f32)
    gids = lax.broadcasted_iota(jnp.int32, (1, B), 1)
    oh = (batch_ref[...] == gids).astype(_f32)
    xa_ref[...] = (jnp.dot(x_ref[...], w1a_ref[...], preferred_element_type=_f32)
                   + jnp.dot(oh, ug, preferred_element_type=_f32))
    xb_ref[...] = jnp.dot(x_ref[...], w1b_ref[...], preferred_element_type=_f32)


def _tc_prep(x, batch2d, u, w1a, w1b, w1u):
    return pl.pallas_call(
        _prep_body,
        grid=(N // BN,),
        in_specs=[
            pl.BlockSpec((BN, DV), lambda i: (i, 0)),
            pl.BlockSpec((BN, 1), lambda i: (i, 0)),
            pl.BlockSpec((B, DU), lambda i: (0, 0)),
            pl.BlockSpec((DV, DE), lambda i: (0, 0)),
            pl.BlockSpec((DV, DE), lambda i: (0, 0)),
            pl.BlockSpec((DU, DE), lambda i: (0, 0)),
        ],
        out_specs=[
            pl.BlockSpec((BN, DE), lambda i: (i, 0)),
            pl.BlockSpec((BN, DE), lambda i: (i, 0)),
        ],
        out_shape=[
            jax.ShapeDtypeStruct((N, DE), _f32),
            jax.ShapeDtypeStruct((N, DE), _f32),
        ],
    )(x, batch2d, u, w1a, w1b, w1u)


def _sc_gather(xa, xb, rowm, colm, eat):
    mesh = plsc.VectorSubcoreMesh(core_axis_name="c", subcore_axis_name="s",
                                  num_cores=NC, num_subcores=NS)

    @functools.partial(
        pl.kernel,
        out_type=(jax.ShapeDtypeStruct((E, DE), _f32),
                  jax.ShapeDtypeStruct((E, DE), _f32),
                  jax.ShapeDtypeStruct((E, DE), _f32)),
        mesh=mesh,
        scratch_types=[
            pltpu.VMEM((2, KC2, CL), jnp.int32),
            pltpu.VMEM((2, KC2, CL), jnp.int32),
            pltpu.VMEM((GE2, DE), _f32),
            pltpu.VMEM((GE2, DE), _f32),
            pltpu.VMEM((DE, GE2), _f32),
            pltpu.VMEM((GE2, DE), _f32),
            pltpu.SemaphoreType.DMA,
            pltpu.SemaphoreType.DMA,
            pltpu.SemaphoreType.DMA,
            pltpu.SemaphoreType.DMA,
            pltpu.SemaphoreType.DMA,
        ],
        compiler_params=pltpu.CompilerParams(use_tc_tiling_on_sc=False, needs_layout_passes=False),
    )
    def k(xa_hbm, xb_hbm, rowm_hbm, colm_hbm, eat_hbm, a_out, b_out, ea_out,
          idx_r, idx_c, a_buf, b_buf, eat_buf, ea_buf,
          sem_a, sem_b, sem_e, sem_o, sem_i):
        c = lax.axis_index("c")
        s = lax.axis_index("s")
        wid = c * NS + s
        jidx = lax.broadcasted_iota(jnp.int32, (16,), 0)
        pltpu.sync_copy(rowm_hbm.at[pl.ds(wid * NCHUNK, KC2)], idx_r.at[0])
        pltpu.sync_copy(colm_hbm.at[pl.ds(wid * NCHUNK, KC2)], idx_c.at[0])
        for g in range(NG2):
            ebase = wid * EW + g * GE2
            for j in range(DE):
                pltpu.async_copy(eat_hbm.at[j, pl.ds(ebase, GE2)],
                                 eat_buf.at[j], sem_e)
            if g > 0:
                pb = wid * EW + (g - 1) * GE2
                pltpu.make_async_copy(a_out.at[pl.ds(pb, GE2)], a_buf, sem_o).wait()
                pltpu.make_async_copy(b_out.at[pl.ds(pb, GE2)], b_buf, sem_o).wait()
                pltpu.make_async_copy(a_out.at[pl.ds(pb, GE2)], ea_buf, sem_o).wait()
                pltpu.make_async_copy(rowm_hbm.at[pl.ds(0, KC2)],
                                      idx_r.at[g % 2], sem_i).wait()
                pltpu.make_async_copy(rowm_hbm.at[pl.ds(0, KC2)],
                                      idx_c.at[g % 2], sem_i).wait()
            if g + 1 < NG2:
                nbase = wid * NCHUNK + (g + 1) * KC2
                pltpu.async_copy(rowm_hbm.at[pl.ds(nbase, KC2)],
                                 idx_r.at[(g + 1) % 2], sem_i)
                pltpu.async_copy(colm_hbm.at[pl.ds(nbase, KC2)],
                                 idx_c.at[(g + 1) % 2], sem_i)

            ir = idx_r.at[g % 2]
            ic = idx_c.at[g % 2]

            def fire(j, carry):
                pltpu.async_copy(xa_hbm.at[ir.at[j]],
                                 a_buf.at[pl.ds(j * CL, CL)], sem_a)
                pltpu.async_copy(xb_hbm.at[ic.at[j]],
                                 b_buf.at[pl.ds(j * CL, CL)], sem_b)
                return carry

            lax.fori_loop(0, KC2, fire, 0)
            pltpu.make_async_copy(eat_hbm.at[:, pl.ds(0, GE2)], eat_buf,
                                  sem_e).wait()

            def tbody(e, carry):
                ev = jnp.broadcast_to(e, (16,)).astype(jnp.int32)
                v = plsc.load_gather(eat_buf, [jidx, ev])
                ea_buf[e, :] = v
                return carry

            lax.fori_loop(0, GE2, tbody, 0)
            pltpu.make_async_copy(a_out.at[pl.ds(ebase, GE2)], a_buf, sem_a).wait()
            pltpu.make_async_copy(b_out.at[pl.ds(ebase, GE2)], b_buf, sem_b).wait()
            pltpu.async_copy(a_buf, a_out.at[pl.ds(ebase, GE2)], sem_o)
            pltpu.async_copy(b_buf, b_out.at[pl.ds(ebase, GE2)], sem_o)
            pltpu.async_copy(ea_buf, ea_out.at[pl.ds(ebase, GE2)], sem_o)
        fb = wid * EW + (NG2 - 1) * GE2
        pltpu.make_async_copy(a_out.at[pl.ds(fb, GE2)], a_buf, sem_o).wait()
        pltpu.make_async_copy(b_out.at[pl.ds(fb, GE2)], b_buf, sem_o).wait()
        pltpu.make_async_copy(a_out.at[pl.ds(fb, GE2)], ea_buf, sem_o).wait()

    return k(xa, xb, rowm, colm, eat)


EQ = E // 8
BQ = 4000

def _edge_body(a_ref, b_ref, ea_ref, w1e_bd_ref, b1_ref, w2_bd_ref, b2_ref,
               out_ref):
    h1 = (a_ref[...] + b_ref[...]
          + jnp.dot(ea_ref[...], w1e_bd_ref[...], preferred_element_type=_f32)
          + b1_ref[...])
    h1 = jnp.maximum(h1, 0.0)
    h2 = jnp.dot(h1, w2_bd_ref[...], preferred_element_type=_f32) + b2_ref[...]
    out_ref[...] = jnp.maximum(h2, 0.0)


def _tc_edge(a_q, b_q, edge_attr, w1e_bd, b1t, w2_bd, b2t):
    blk = lambda i: (i, 0)
    full = lambda i: (0, 0)
    return pl.pallas_call(
        _edge_body,
        grid=(EQ // BQ,),
        in_specs=[
            pl.BlockSpec((BQ, 128), blk),
            pl.BlockSpec((BQ, 128), blk),
            pl.BlockSpec((BQ, 128), blk),
            pl.BlockSpec((128, 128), full),
            pl.BlockSpec((1, 128), full),
            pl.BlockSpec((128, 128), full),
            pl.BlockSpec((1, 128), full),
        ],
        out_specs=pl.BlockSpec((BQ, 128), blk),
        out_shape=jax.ShapeDtypeStruct((EQ, 128), _f32),
    )(a_q, b_q, edge_attr, w1e_bd, b1t, w2_bd, b2t)


def _sc_scatter(e2, rowm, zeros_nd, ones_cl):
    mesh = plsc.VectorSubcoreMesh(core_axis_name="c", subcore_axis_name="s",
                                  num_cores=NC, num_subcores=NS)

    @functools.partial(
        pl.kernel,
        out_type=(jax.ShapeDtypeStruct((NC, N, 128), _f32),
                  jax.ShapeDtypeStruct((NC, N, 128), _f32),
                  jax.ShapeDtypeStruct((DE, E), _f32)),
        mesh=mesh,
        scratch_types=[
            pltpu.VMEM_SHARED((N, DE), _f32),
            pltpu.VMEM_SHARED((N, DE), _f32),
            pltpu.VMEM((2, KC, CL), jnp.int32),
            pltpu.VMEM((2, GE, DE), _f32),
            pltpu.VMEM((DE, GE), _f32),
            pltpu.VMEM((CL, DE), _f32),
            pltpu.SemaphoreType.DMA,
            pltpu.SemaphoreType.DMA,
            pltpu.SemaphoreType.DMA,
            pltpu.SemaphoreType.DMA,
        ],
        compiler_params=pltpu.CompilerParams(use_tc_tiling_on_sc=False, needs_layout_passes=False),
    )
    def k(e2_hbm, rowm_hbm, zeros_hbm, ones_hbm, sum_out, cnt_out, e2t_out,
          sum_acc, cnt_acc, idx_v, pay_v, pay_t, ones_v,
          sem_s, sem_c, sem_t, sem_p):
        c = lax.axis_index("c")
        s = lax.axis_index("s")
        wid = c * NS + s
        jidx = lax.broadcasted_iota(jnp.int32, (16,), 0)

        @pl.when(s == 0)
        def _():
            pltpu.sync_copy(zeros_hbm, sum_acc)
            pltpu.sync_copy(zeros_hbm, cnt_acc)

        pltpu.sync_copy(ones_hbm, ones_v)
        plsc.subcore_barrier()

        pltpu.sync_copy(rowm_hbm.at[pl.ds(wid * NCHUNK, KC)], idx_v.at[0])
        pltpu.sync_copy(e2_hbm.at[pl.ds(wid * EW, GE)], pay_v.at[0])
        for g in range(NG):
            ebase = wid * EW + g * GE
            if g > 0:
                pltpu.make_async_copy(e2_hbm.at[pl.ds(0, GE)],
                                      pay_v.at[g % 2], sem_p).wait()
                pltpu.make_async_copy(rowm_hbm.at[pl.ds(0, KC)],
                                      idx_v.at[g % 2], sem_p).wait()
                pltpu.make_async_copy(e2t_out.at[:, pl.ds(0, GE)], pay_t,
                                      sem_t).wait()
            if g + 1 < NG:
                nc_ = wid * NCHUNK + (g + 1) * KC
                ne_ = wid * EW + (g + 1) * GE
                pltpu.async_copy(rowm_hbm.at[pl.ds(nc_, KC)],
                                 idx_v.at[(g + 1) % 2], sem_p)
                pltpu.async_copy(e2_hbm.at[pl.ds(ne_, GE)],
                                 pay_v.at[(g + 1) % 2], sem_p)

            pv = pay_v.at[g % 2]
            iv = idx_v.at[g % 2]

            def tbody(e, carry):
                ev = jnp.broadcast_to(e, (16,)).astype(jnp.int32)
                v = pv[e, :]
                plsc.store_scatter(pay_t, [jidx, ev], v)
                return carry

            lax.fori_loop(0, GE, tbody, 0)
            for j in range(DE):
                pltpu.async_copy(pay_t.at[j],
                                 e2t_out.at[j, pl.ds(ebase, GE)], sem_t)

            def fire(j, carry):
                pltpu.async_copy(pv.at[pl.ds(j * CL, CL)],
                                 sum_acc.at[iv.at[j]], sem_s, add=True)
                pltpu.async_copy(ones_v,
                                 cnt_acc.at[iv.at[j]], sem_c, add=True)
                return carry

            lax.fori_loop(0, KC, fire, 0)
            pltpu.make_async_copy(zeros_hbm.at[pl.ds(0, GE)], pay_v.at[0],
                                  sem_s).wait()
            pltpu.make_async_copy(zeros_hbm.at[pl.ds(0, GE)], pay_v.at[0],
                                  sem_c).wait()

        pltpu.make_async_copy(e2t_out.at[:, pl.ds(0, GE)], pay_t, sem_t).wait()
        plsc.subcore_barrier()

        @pl.when(s < NWR)
        def _():
            pltpu.sync_copy(sum_acc.at[pl.ds(s * NR, NR)],
                            sum_out.at[c, pl.ds(s * NR, NR), pl.ds(0, DE)])
            pltpu.sync_copy(cnt_acc.at[pl.ds(s * NR, NR)],
                            cnt_out.at[c, pl.ds(s * NR, NR), pl.ds(0, DE)])

    return k(e2, rowm, zeros_nd, ones_cl)


def _node_body(x_ref, batch_ref, sums_ref, cnts_ref, u_ref,
               wvx_ref, wve_ref, wvu_ref, b1_ref, w2_ref, b2_ref,
               wua_ref, wub_ref, wuc_ref, ub1_ref, uw2_ref, ub2_ref,
               x2_ref, u2_ref,
               enum_acc, eden_acc, vnum_acc, vden_acc):
    i = pl.program_id(0)
    nsteps = pl.num_programs(0)
    gids = lax.broadcasted_iota(jnp.int32, (1, B), 1)
    oh = (batch_ref[...] == gids).astype(_f32)
    sum_blk = sums_ref[0, :, :DE] + sums_ref[1, :, :DE]
    cnt_blk = cnts_ref[0, :, :DE] + cnts_ref[1, :, :DE]
    e_aggr = sum_blk / jnp.maximum(cnt_blk, 1.0)

    uvw = jnp.dot(u_ref[...], wvu_ref[...], preferred_element_type=_f32)
    h = (jnp.dot(x_ref[...], wvx_ref[...], preferred_element_type=_f32)
         + jnp.dot(e_aggr, wve_ref[...], preferred_element_type=_f32)
         + jnp.dot(oh, uvw, preferred_element_type=_f32)
         + b1_ref[...])
    h = jnp.maximum(h, 0.0)
    x2 = jnp.dot(h, w2_ref[...], preferred_element_type=_f32) + b2_ref[...]
    x2 = jnp.maximum(x2, 0.0)
    x2_ref[...] = x2

    @pl.when(i == 0)
    def _():
        enum_acc[...] = jnp.zeros((B, DE), _f32)
        eden_acc[...] = jnp.zeros((B, DE), _f32)
        vnum_acc[...] = jnp.zeros((B, DV), _f32)
        vden_acc[...] = jnp.zeros((B, DE), _f32)

    contract0 = (((0,), (0,)), ((), ()))
    enum_acc[...] += lax.dot_general(oh, sum_blk, contract0,
                                     preferred_element_type=_f32)
    eden_acc[...] += lax.dot_general(oh, cnt_blk, contract0,
                                     preferred_element_type=_f32)
    vnum_acc[...] += lax.dot_general(oh, x2, contract0,
                                     preferred_element_type=_f32)
    vden_acc[...] += lax.dot_general(oh, jnp.ones((BN, DE), _f32), contract0,
                                     preferred_element_type=_f32)

    @pl.when(i == nsteps - 1)
    def _():
        e_mean = enum_acc[...] / jnp.maximum(eden_acc[...], 1.0)
        v_mean = vnum_acc[...] / jnp.maximum(vden_acc[...][:, :1], 1.0)
        hu = (jnp.dot(u_ref[...], wua_ref[...], preferred_element_type=_f32)
              + jnp.dot(e_mean, wub_ref[...], preferred_element_type=_f32)
              + jnp.dot(v_mean, wuc_ref[...], preferred_element_type=_f32)
              + ub1_ref[...])
        hu = jnp.maximum(hu, 0.0)
        u2 = jnp.dot(hu, uw2_ref[...], preferred_element_type=_f32) + ub2_ref[...]
        u2_ref[...] = jnp.maximum(u2, 0.0)


def _tc_node(x, batch2d, sums, cnts, u,
             wvx, wve, wvu, pv_b1, pv_W2, pv_b2,
             wua, wub, wuc, pu_b1, pu_W2, pu_b2):
    full = lambda i: (0, 0)
    return pl.pallas_call(
        _node_body,
        grid=(N // BN,),
        in_specs=[
            pl.BlockSpec((BN, DV), lambda i: (i, 0)),
            pl.BlockSpec((BN, 1), lambda i: (i, 0)),
            pl.BlockSpec((NC, BN, 128), lambda i: (0, i, 0)),
            pl.BlockSpec((NC, BN, 128), lambda i: (0, i, 0)),
            pl.BlockSpec((B, DU), full),
            pl.BlockSpec((DV, DV), full),
            pl.BlockSpec((DE, DV), full),
            pl.BlockSpec((DU, DV), full),
            pl.BlockSpec((1, DV), full),
            pl.BlockSpec((DV, DV), full),
            pl.BlockSpec((1, DV), full),
            pl.BlockSpec((DU, DU), full),
            pl.BlockSpec((DE, DU), full),
            pl.BlockSpec((DV, DU), full),
            pl.BlockSpec((1, DU), full),
            pl.BlockSpec((DU, DU), full),
            pl.BlockSpec((1, DU), full),
        ],
        out_specs=[
            pl.BlockSpec((BN, DV), lambda i: (i, 0)),
            pl.BlockSpec((B, DU), full),
        ],
        out_shape=[
            jax.ShapeDtypeStruct((N, DV), _f32),
            jax.ShapeDtypeStruct((B, DU), _f32),
        ],
        scratch_shapes=[
            pltpu.VMEM((B, DE), _f32),
            pltpu.VMEM((B, DE), _f32),
            pltpu.VMEM((B, DV), _f32),
            pltpu.VMEM((B, DE), _f32),
        ],
    )(x, batch2d, sums, cnts, u,
      wvx, wve, wvu, pv_b1, pv_W2, pv_b2,
      wua, wub, wuc, pu_b1, pu_W2, pu_b2)


def kernel(x, edge_index, edge_attr, u, batch,
           pe_W1, pe_b1, pe_W2, pe_b2,
           pv_W1, pv_b1, pv_W2, pv_b2,
           pu_W1, pu_b1, pu_W2, pu_b2):
    row = edge_index[0].astype(jnp.int32)
    col = edge_index[1].astype(jnp.int32)
    batch2d = batch.astype(jnp.int32).reshape(N, 1)
    rowm = row.reshape(NW * NCHUNK, CL)
    colm = col.reshape(NW * NCHUNK, CL)

    w1a = pe_W1[:DV]
    w1b = pe_W1[DV:2 * DV]
    w1e = pe_W1[2 * DV:2 * DV + DE]
    w1u = pe_W1[2 * DV + DE:]

    xa, xb = _tc_prep(x, batch2d, u, w1a, w1b, w1u)
    a_g, b_g, ea_lin = _sc_gather(xa, xb, rowm, colm, edge_attr.T)
    eye8 = jnp.eye(8, dtype=_f32)
    e2_q = _tc_edge(a_g.reshape(EQ, 128), b_g.reshape(EQ, 128),
                    ea_lin.reshape(EQ, 128),
                    jnp.kron(eye8, w1e), jnp.tile(pe_b1, 8).reshape(1, 128),
                    jnp.kron(eye8, pe_W2), jnp.tile(pe_b2, 8).reshape(1, 128))
    e2 = e2_q.reshape(E, DE)
    zeros_nd = jnp.zeros((N, DE), _f32)
    ones_cl = jnp.ones((CL, DE), _f32)
    sums, cnts, e2t = _sc_scatter(e2, rowm, zeros_nd, ones_cl)

    wvx = pv_W1[:DV]
    wve = pv_W1[DV:DV + DE]
    wvu = pv_W1[DV + DE:]
    wua = pu_W1[:DU]
    wub = pu_W1[DU:DU + DE]
    wuc = pu_W1[DU + DE:]
    x2, u2 = _tc_node(x, batch2d, sums, cnts, u,
                      wvx, wve, wvu, pv_b1.reshape(1, DV), pv_W2,
                      pv_b2.reshape(1, DV),
                      wua, wub, wuc, pu_b1.reshape(1, DU), pu_W2,
                      pu_b2.reshape(1, DU))
    return (x2, e2t.T, u2)
